# Initial kernel scaffold; baseline (speedup 1.0000x reference)
import jax
import jax.numpy as jnp
from jax.experimental import pallas as pl


def kernel(atom_features, index_list, W_att, b_att, W_out, b_out):
    raise NotImplementedError("write your pallas kernel here")


# speedup vs baseline: 7.6167x; 7.6167x over previous
import functools

import jax
import jax.numpy as jnp
from jax.experimental import pallas as pl
from jax.experimental.pallas import tpu as pltpu

D = 2048
N_TOK = 32768
B = 16
R = 1024
NEG = -1e30


def _body(idx_ref, watt_ref, batt_ref, bout_ref, a_ref, wout_ref, out_ref,
          m_ref, l_ref, acc_ref, *, nb):
    i = pl.program_id(0)

    @pl.when(i == 0)
    def _init():
        m_ref[...] = jnp.full_like(m_ref, NEG)
        l_ref[...] = jnp.zeros_like(l_ref)
        acc_ref[...] = jnp.zeros_like(acc_ref)

    a = a_ref[...]
    s = jax.lax.dot_general(
        a, watt_ref[...], (((1,), (0,)), ((), ())),
        preferred_element_type=jnp.float32) + batt_ref[0, 0]

    pos = i * R + jax.lax.broadcasted_iota(jnp.int32, (R, B), 0)
    st = idx_ref[...][:, 0][None, :]
    en = idx_ref[...][:, 1][None, :]
    mask = (pos >= st) & (pos <= en)

    sb = jnp.where(mask, s, NEG)
    bm = jnp.max(sb, axis=0)
    m_old = m_ref[0]
    m_new = jnp.maximum(m_old, bm)
    alpha = jnp.exp(m_old - m_new)
    e = jnp.where(mask, jnp.exp(sb - m_new[None, :]), 0.0)
    l_ref[0] = alpha * l_ref[0] + jnp.sum(e, axis=0)
    m_ref[0] = m_new
    acc_ref[...] = acc_ref[...] * alpha[:, None] + jax.lax.dot_general(
        e, a, (((0,), (0,)), ((), ())),
        preferred_element_type=jnp.float32)

    @pl.when(i == nb - 1)
    def _fin():
        pooled = acc_ref[...] / l_ref[0][:, None]
        out_ref[...] = jax.lax.dot_general(
            pooled, wout_ref[...], (((1,), (0,)), ((), ())),
            preferred_element_type=jnp.float32) + bout_ref[...]


@jax.jit
def kernel(atom_features, index_list, W_att, b_att, W_out, b_out):
    nb = N_TOK // R
    return pl.pallas_call(
        functools.partial(_body, nb=nb),
        grid=(nb,),
        in_specs=[
            pl.BlockSpec((B, 2), lambda i: (0, 0)),
            pl.BlockSpec((D, 1), lambda i: (0, 0)),
            pl.BlockSpec((1, 1), lambda i: (0, 0)),
            pl.BlockSpec((1, D), lambda i: (0, 0)),
            pl.BlockSpec((R, D), lambda i: (i, 0)),
            pl.BlockSpec((D, D), lambda i: (0, 0)),
        ],
        out_specs=pl.BlockSpec((B, D), lambda i: (0, 0)),
        out_shape=jax.ShapeDtypeStruct((B, D), jnp.float32),
        scratch_shapes=[
            pltpu.VMEM((1, B), jnp.float32),
            pltpu.VMEM((1, B), jnp.float32),
            pltpu.VMEM((B, D), jnp.float32),
        ],
    )(index_list.astype(jnp.int32), W_att, b_att.reshape(1, 1),
      b_out.reshape(1, D), atom_features, W_out)
